# baseline (speedup 1.0000x reference)
import functools

import jax
import jax.numpy as jnp
from jax import lax
from jax.experimental import pallas as pl
from jax.experimental.pallas import tpu as pltpu
from jax.experimental.pallas import tpu_sc as plsc

F32 = jnp.float32
NSUB = 16
NCORE = 2
BATCH = 64
DEGW = 16


def _sc_degree(dstp, zdeg, nacc, eps):
  nb = eps // BATCH
  nw = NCORE * NSUB
  mesh = plsc.VectorSubcoreMesh(core_axis_name="c", subcore_axis_name="s")

  @functools.partial(
      pl.kernel,
      out_type=jax.ShapeDtypeStruct((nw, nacc), F32),
      mesh=mesh,
      scratch_types=[
          pltpu.VMEM((BATCH,), jnp.int32),
          pltpu.VMEM((nacc,), F32),
      ],
      compiler_params=pltpu.CompilerParams(needs_layout_passes=False),
  )
  def deg_kernel(dst_hbm, z_hbm, out_hbm, idxb, hist):
    c = lax.axis_index("c")
    s = lax.axis_index("s")
    w = c * NSUB + s
    pltpu.sync_copy(z_hbm, hist)
    base0 = w * eps

    @pl.loop(0, nb)
    def _(i):
      pltpu.sync_copy(dst_hbm.at[pl.ds(base0 + i * BATCH, BATCH)], idxb)
      for j in range(BATCH // 16):
        d16 = idxb[pl.ds(j * 16, 16)]
        cnt, last = plsc.scan_count(d16)
        old = plsc.load_gather(hist, [d16], mask=last)
        plsc.store_scatter(hist, [d16], old + cnt.astype(F32), mask=last)

    pltpu.sync_copy(hist, out_hbm.at[w])

  return deg_kernel(dstp, zdeg)


def _sc_prop(y, srcp, dstp, zhbm, nacc, eps, d, untiled=False):
  nb = eps // BATCH
  rows_sub = nacc // NSUB
  mesh = plsc.VectorSubcoreMesh(core_axis_name="c", subcore_axis_name="s")
  params = (pltpu.CompilerParams(use_tc_tiling_on_sc=False)
            if untiled else None)

  @functools.partial(
      pl.kernel,
      out_type=jax.ShapeDtypeStruct((NCORE, nacc, d), F32),
      mesh=mesh,
      compiler_params=params,
      scratch_types=(
          [pltpu.VMEM((BATCH,), jnp.int32)] * 4
          + [pltpu.VMEM((BATCH,), jnp.int32)] * 4
          + [pltpu.VMEM((BATCH, d), F32)] * 4
          + [pltpu.VMEM_SHARED((nacc, d), F32)]
          + [pltpu.SemaphoreType.DMA] * 8
      ),
  )
  def prop_kernel(y_hbm, src_hbm, dst_hbm, z_hbm, out_hbm, *scratch):
    srcb = scratch[0:4]
    dstb = scratch[4:8]
    rows = scratch[8:12]
    acc = scratch[12]
    semg = scratch[13:17]
    sems = scratch[17:21]
    c = lax.axis_index("c")
    s = lax.axis_index("s")
    w = c * NSUB + s
    pltpu.sync_copy(z_hbm.at[pl.ds(s * rows_sub, rows_sub)],
                    acc.at[pl.ds(s * rows_sub, rows_sub)])
    plsc.subcore_barrier()
    base0 = w * eps

    def load_and_gather(i, q):
      b0 = base0 + i * BATCH
      pltpu.sync_copy(src_hbm.at[pl.ds(b0, BATCH)], srcb[q])
      pltpu.sync_copy(dst_hbm.at[pl.ds(b0, BATCH)], dstb[q])
      pltpu.async_copy(y_hbm.at[srcb[q]], rows[q], semg[q])

    def wait_gather(q):
      pltpu.make_async_copy(y_hbm.at[srcb[q]], rows[q], semg[q]).wait()

    def start_scatter(q):
      pltpu.async_copy(rows[q], acc.at[dstb[q]], sems[q], add=True)

    def wait_scatter(q):
      pltpu.make_async_copy(rows[q], acc.at[dstb[q]], sems[q]).wait()

    load_and_gather(0, 0)
    load_and_gather(1, 1)

    @pl.loop(0, nb // 4)
    def _(k):
      for j in range(4):
        i = 4 * k + j
        q = j
        q2 = (j + 2) % 4
        wait_gather(q)
        start_scatter(q)
        if j < 2:
          @pl.when(jnp.logical_and(k >= 1, i + 2 < nb))
          def _():
            wait_scatter(q2)
            load_and_gather(i + 2, q2)

          @pl.when(k == 0)
          def _():
            load_and_gather(i + 2, q2)
        else:
          @pl.when(i + 2 < nb)
          def _():
            wait_scatter(q2)
            load_and_gather(i + 2, q2)

    for q in range(4):
      wait_scatter(q)
    plsc.subcore_barrier()
    pltpu.sync_copy(acc.at[pl.ds(s * rows_sub, rows_sub)],
                    out_hbm.at[c, pl.ds(s * rows_sub, rows_sub)])

  return prop_kernel(y, srcp, dstp, zhbm)


def _tc_matmul(x, w1, n, nfeat, nhid, rb):
  grid = n // rb

  def body(x_ref, w_ref, h_ref):
    h_ref[...] = jnp.dot(x_ref[...], w_ref[...], preferred_element_type=F32)

  return pl.pallas_call(
      body,
      grid=(grid,),
      in_specs=[
          pl.BlockSpec((rb, nfeat), lambda i: (i, 0)),
          pl.BlockSpec((nfeat, nhid), lambda i: (0, 0)),
      ],
      out_specs=pl.BlockSpec((rb, nhid), lambda i: (i, 0)),
      out_shape=jax.ShapeDtypeStruct((n, nhid), F32),
  )(x, w1)


def _tc_scale1(h1, degp, n, nhid, rb):
  grid = n // rb
  nw = degp.shape[1]

  def body(h_ref, dp_ref, y_ref, dinv_ref):
    ones = jnp.ones((nw, 1), F32)
    deg = jnp.dot(dp_ref[...], ones, preferred_element_type=F32) + 1.0
    dinv = lax.rsqrt(jnp.maximum(deg, 1.0))
    y_ref[...] = dinv * h_ref[...]
    dinv_ref[...] = dinv

  return pl.pallas_call(
      body,
      grid=(grid,),
      in_specs=[
          pl.BlockSpec((rb, nhid), lambda i: (i, 0)),
          pl.BlockSpec((rb, nw), lambda i: (i, 0)),
      ],
      out_specs=[
          pl.BlockSpec((rb, nhid), lambda i: (i, 0)),
          pl.BlockSpec((rb, 1), lambda i: (i, 0)),
      ],
      out_shape=[
          jax.ShapeDtypeStruct((n, nhid), F32),
          jax.ShapeDtypeStruct((n, 1), F32),
      ],
  )(h1, degp)


def _tc_layer2(p0, p1, y1, dinv, b1r, w2p, n, nhid, ncp, rb):
  grid = n // rb

  def body(p0_ref, p1_ref, y1_ref, dinv_ref, b1_ref, w_ref, y2_ref):
    a = p0_ref[...] + p1_ref[...] + y1_ref[...]
    t = jnp.maximum(dinv_ref[...] * a + b1_ref[...], 0.0)
    h2 = jnp.dot(t, w_ref[...], preferred_element_type=F32)
    y2_ref[...] = dinv_ref[...] * h2

  return pl.pallas_call(
      body,
      grid=(grid,),
      in_specs=[
          pl.BlockSpec((rb, nhid), lambda i: (i, 0)),
          pl.BlockSpec((rb, nhid), lambda i: (i, 0)),
          pl.BlockSpec((rb, nhid), lambda i: (i, 0)),
          pl.BlockSpec((rb, 1), lambda i: (i, 0)),
          pl.BlockSpec((1, nhid), lambda i: (0, 0)),
          pl.BlockSpec((nhid, ncp), lambda i: (0, 0)),
      ],
      out_specs=pl.BlockSpec((rb, ncp), lambda i: (i, 0)),
      out_shape=jax.ShapeDtypeStruct((n, ncp), F32),
  )(p0, p1, y1, dinv, b1r, w2p)


def _tc_out(p0, p1, y2, dinv, b2r, n, ncp, ncls, rb):
  grid = n // rb

  def body(p0_ref, p1_ref, y2_ref, dinv_ref, b2_ref, out_ref):
    v = dinv_ref[...] * (p0_ref[...] + p1_ref[...] + y2_ref[...]) + b2_ref[...]
    out_ref[...] = v[:, :ncls]

  return pl.pallas_call(
      body,
      grid=(grid,),
      in_specs=[
          pl.BlockSpec((rb, ncp), lambda i: (i, 0)),
          pl.BlockSpec((rb, ncp), lambda i: (i, 0)),
          pl.BlockSpec((rb, ncp), lambda i: (i, 0)),
          pl.BlockSpec((rb, 1), lambda i: (i, 0)),
          pl.BlockSpec((1, ncp), lambda i: (0, 0)),
      ],
      out_specs=pl.BlockSpec((rb, ncls), lambda i: (i, 0)),
      out_shape=jax.ShapeDtypeStruct((n, ncls), F32),
  )(p0, p1, y2, dinv, b2r)


def kernel(x, edge_index, W1, b1, W2, b2):
  n, nfeat = x.shape
  nhid = W1.shape[1]
  ncls = W2.shape[1]
  e = edge_index.shape[1]
  nw = NCORE * NSUB

  eps = -(-e // (nw * 4 * BATCH)) * 4 * BATCH
  etot = eps * nw
  nacc = -(-(n + BATCH) // 128) * 128
  ncp = 128
  rb = 1000

  ei = edge_index.astype(jnp.int32)
  pad = etot - e
  spread = jnp.arange(pad, dtype=jnp.int32) % BATCH
  srcp = jnp.concatenate([ei[0], spread])
  dstp = jnp.concatenate([ei[1], n + spread])

  zdeg = jnp.zeros((nacc,), F32)
  z1 = jnp.zeros((nacc, nhid), F32)
  z2 = jnp.zeros((nacc, ncp), F32)
  w2p = jnp.pad(W2, ((0, 0), (0, ncp - ncls)))
  b1r = b1.reshape(1, nhid)
  b2r = jnp.pad(b2, (0, ncp - ncls)).reshape(1, ncp)

  degp = _sc_degree(dstp, zdeg, nacc, eps).T
  h1 = _tc_matmul(x, W1, n, nfeat, nhid, rb)
  y1, dinv = _tc_scale1(h1, degp, n, nhid, rb)
  p1 = _sc_prop(y1, srcp, dstp, z1, nacc, eps, nhid)
  y2 = _tc_layer2(p1[0], p1[1], y1, dinv, b1r, w2p, n, nhid, ncp, rb)
  p2 = _sc_prop(y2, srcp, dstp, z2, nacc, eps, ncp)
  return _tc_out(p2[0], p2[1], y2, dinv, b2r, n, ncp, ncls, rb)


# speedup vs baseline: 1.1161x; 1.1161x over previous
import functools

import jax
import jax.numpy as jnp
from jax import lax
from jax.experimental import pallas as pl
from jax.experimental.pallas import tpu as pltpu
from jax.experimental.pallas import tpu_sc as plsc

F32 = jnp.float32
NSUB = 16
NCORE = 2
BATCH = 80
DEGW = 16


def _sc_degree(dstp, zdeg, nacc, eps):
  nb = eps // BATCH
  nw = NCORE * NSUB
  mesh = plsc.VectorSubcoreMesh(core_axis_name="c", subcore_axis_name="s")

  @functools.partial(
      pl.kernel,
      out_type=jax.ShapeDtypeStruct((nw, nacc), F32),
      mesh=mesh,
      scratch_types=[
          pltpu.VMEM((BATCH,), jnp.int32),
          pltpu.VMEM((nacc,), F32),
      ],
      compiler_params=pltpu.CompilerParams(needs_layout_passes=False),
  )
  def deg_kernel(dst_hbm, z_hbm, out_hbm, idxb, hist):
    c = lax.axis_index("c")
    s = lax.axis_index("s")
    w = c * NSUB + s
    pltpu.sync_copy(z_hbm, hist)
    base0 = w * eps

    @pl.loop(0, nb)
    def _(i):
      pltpu.sync_copy(dst_hbm.at[pl.ds(base0 + i * BATCH, BATCH)], idxb)
      for j in range(BATCH // 16):
        d16 = idxb[pl.ds(j * 16, 16)]
        cnt, last = plsc.scan_count(d16)
        old = plsc.load_gather(hist, [d16], mask=last)
        plsc.store_scatter(hist, [d16], old + cnt.astype(F32), mask=last)

    pltpu.sync_copy(hist, out_hbm.at[w])

  return deg_kernel(dstp, zdeg)


def _sc_prop(y, srcp, dstp, zhbm, nacc, eps, d, untiled=False):
  nb = eps // BATCH
  rows_sub = nacc // NSUB
  mesh = plsc.VectorSubcoreMesh(core_axis_name="c", subcore_axis_name="s")
  params = (pltpu.CompilerParams(use_tc_tiling_on_sc=False)
            if untiled else None)

  @functools.partial(
      pl.kernel,
      out_type=jax.ShapeDtypeStruct((NCORE, nacc, d), F32),
      mesh=mesh,
      compiler_params=params,
      scratch_types=(
          [pltpu.VMEM((BATCH,), jnp.int32)] * 4
          + [pltpu.VMEM((BATCH,), jnp.int32)] * 4
          + [pltpu.VMEM((BATCH, d), F32)] * 4
          + [pltpu.VMEM_SHARED((nacc, d), F32)]
          + [pltpu.SemaphoreType.DMA] * 8
      ),
  )
  def prop_kernel(y_hbm, src_hbm, dst_hbm, z_hbm, out_hbm, *scratch):
    srcb = scratch[0:4]
    dstb = scratch[4:8]
    rows = scratch[8:12]
    acc = scratch[12]
    semg = scratch[13:17]
    sems = scratch[17:21]
    c = lax.axis_index("c")
    s = lax.axis_index("s")
    w = c * NSUB + s
    pltpu.sync_copy(z_hbm.at[pl.ds(s * rows_sub, rows_sub)],
                    acc.at[pl.ds(s * rows_sub, rows_sub)])
    plsc.subcore_barrier()
    base0 = w * eps

    def load_and_gather(i, q):
      b0 = base0 + i * BATCH
      pltpu.sync_copy(src_hbm.at[pl.ds(b0, BATCH)], srcb[q])
      pltpu.sync_copy(dst_hbm.at[pl.ds(b0, BATCH)], dstb[q])
      pltpu.async_copy(y_hbm.at[srcb[q]], rows[q], semg[q])

    def wait_gather(q):
      pltpu.make_async_copy(y_hbm.at[srcb[q]], rows[q], semg[q]).wait()

    def start_scatter(q):
      pltpu.async_copy(rows[q], acc.at[dstb[q]], sems[q], add=True)

    def wait_scatter(q):
      pltpu.make_async_copy(rows[q], acc.at[dstb[q]], sems[q]).wait()

    load_and_gather(0, 0)
    load_and_gather(1, 1)

    @pl.loop(0, nb // 4)
    def _(k):
      for j in range(4):
        i = 4 * k + j
        q = j
        q2 = (j + 2) % 4
        wait_gather(q)
        start_scatter(q)
        if j < 2:
          @pl.when(jnp.logical_and(k >= 1, i + 2 < nb))
          def _():
            wait_scatter(q2)
            load_and_gather(i + 2, q2)

          @pl.when(k == 0)
          def _():
            load_and_gather(i + 2, q2)
        else:
          @pl.when(i + 2 < nb)
          def _():
            wait_scatter(q2)
            load_and_gather(i + 2, q2)

    for q in range(4):
      wait_scatter(q)
    plsc.subcore_barrier()
    pltpu.sync_copy(acc.at[pl.ds(s * rows_sub, rows_sub)],
                    out_hbm.at[c, pl.ds(s * rows_sub, rows_sub)])

  return prop_kernel(y, srcp, dstp, zhbm)


def _tc_matmul(x, w1, n, nfeat, nhid, rb):
  grid = n // rb

  def body(x_ref, w_ref, h_ref):
    h_ref[...] = jnp.dot(x_ref[...], w_ref[...], preferred_element_type=F32)

  return pl.pallas_call(
      body,
      grid=(grid,),
      in_specs=[
          pl.BlockSpec((rb, nfeat), lambda i: (i, 0)),
          pl.BlockSpec((nfeat, nhid), lambda i: (0, 0)),
      ],
      out_specs=pl.BlockSpec((rb, nhid), lambda i: (i, 0)),
      out_shape=jax.ShapeDtypeStruct((n, nhid), F32),
  )(x, w1)


def _tc_scale1(h1, degp, n, nhid, rb):
  grid = n // rb
  nw = degp.shape[1]

  def body(h_ref, dp_ref, y_ref, dinv_ref):
    ones = jnp.ones((nw, 1), F32)
    deg = jnp.dot(dp_ref[...], ones, preferred_element_type=F32) + 1.0
    dinv = lax.rsqrt(jnp.maximum(deg, 1.0))
    y_ref[...] = dinv * h_ref[...]
    dinv_ref[...] = dinv

  return pl.pallas_call(
      body,
      grid=(grid,),
      in_specs=[
          pl.BlockSpec((rb, nhid), lambda i: (i, 0)),
          pl.BlockSpec((rb, nw), lambda i: (i, 0)),
      ],
      out_specs=[
          pl.BlockSpec((rb, nhid), lambda i: (i, 0)),
          pl.BlockSpec((rb, 1), lambda i: (i, 0)),
      ],
      out_shape=[
          jax.ShapeDtypeStruct((n, nhid), F32),
          jax.ShapeDtypeStruct((n, 1), F32),
      ],
  )(h1, degp)


def _tc_layer2(p0, p1, y1, dinv, b1r, w2p, n, nhid, ncp, rb):
  grid = n // rb

  def body(p0_ref, p1_ref, y1_ref, dinv_ref, b1_ref, w_ref, y2_ref):
    a = p0_ref[...] + p1_ref[...] + y1_ref[...]
    t = jnp.maximum(dinv_ref[...] * a + b1_ref[...], 0.0)
    h2 = jnp.dot(t, w_ref[...], preferred_element_type=F32)
    y2_ref[...] = dinv_ref[...] * h2

  return pl.pallas_call(
      body,
      grid=(grid,),
      in_specs=[
          pl.BlockSpec((rb, nhid), lambda i: (i, 0)),
          pl.BlockSpec((rb, nhid), lambda i: (i, 0)),
          pl.BlockSpec((rb, nhid), lambda i: (i, 0)),
          pl.BlockSpec((rb, 1), lambda i: (i, 0)),
          pl.BlockSpec((1, nhid), lambda i: (0, 0)),
          pl.BlockSpec((nhid, ncp), lambda i: (0, 0)),
      ],
      out_specs=pl.BlockSpec((rb, ncp), lambda i: (i, 0)),
      out_shape=jax.ShapeDtypeStruct((n, ncp), F32),
  )(p0, p1, y1, dinv, b1r, w2p)


def _tc_out(p0, p1, y2, dinv, b2r, n, ncp, ncls, rb):
  grid = n // rb

  def body(p0_ref, p1_ref, y2_ref, dinv_ref, b2_ref, out_ref):
    v = dinv_ref[...] * (p0_ref[...] + p1_ref[...] + y2_ref[...]) + b2_ref[...]
    out_ref[...] = v[:, :ncls]

  return pl.pallas_call(
      body,
      grid=(grid,),
      in_specs=[
          pl.BlockSpec((rb, ncp), lambda i: (i, 0)),
          pl.BlockSpec((rb, ncp), lambda i: (i, 0)),
          pl.BlockSpec((rb, ncp), lambda i: (i, 0)),
          pl.BlockSpec((rb, 1), lambda i: (i, 0)),
          pl.BlockSpec((1, ncp), lambda i: (0, 0)),
      ],
      out_specs=pl.BlockSpec((rb, ncls), lambda i: (i, 0)),
      out_shape=jax.ShapeDtypeStruct((n, ncls), F32),
  )(p0, p1, y2, dinv, b2r)


def kernel(x, edge_index, W1, b1, W2, b2):
  n, nfeat = x.shape
  nhid = W1.shape[1]
  ncls = W2.shape[1]
  e = edge_index.shape[1]
  nw = NCORE * NSUB

  eps = -(-e // (nw * 4 * BATCH)) * 4 * BATCH
  etot = eps * nw
  nacc = -(-(n + BATCH) // 128) * 128
  ncp = 128
  rb = 1000

  ei = edge_index.astype(jnp.int32)
  pad = etot - e
  spread = jnp.arange(pad, dtype=jnp.int32) % BATCH
  srcp = jnp.concatenate([ei[0], spread])
  dstp = jnp.concatenate([ei[1], n + spread])

  zdeg = jnp.zeros((nacc,), F32)
  z1 = jnp.zeros((nacc, nhid), F32)
  z2 = jnp.zeros((nacc, ncp), F32)
  w2p = jnp.pad(W2, ((0, 0), (0, ncp - ncls)))
  b1r = b1.reshape(1, nhid)
  b2r = jnp.pad(b2, (0, ncp - ncls)).reshape(1, ncp)

  degp = _sc_degree(dstp, zdeg, nacc, eps).T
  h1 = _tc_matmul(x, W1, n, nfeat, nhid, rb)
  y1, dinv = _tc_scale1(h1, degp, n, nhid, rb)
  p1 = _sc_prop(y1, srcp, dstp, z1, nacc, eps, nhid)
  y2 = _tc_layer2(p1[0], p1[1], y1, dinv, b1r, w2p, n, nhid, ncp, rb)
  p2 = _sc_prop(y2, srcp, dstp, z2, nacc, eps, ncp)
  return _tc_out(p2[0], p2[1], y2, dinv, b2r, n, ncp, ncls, rb)


# speedup vs baseline: 1.1582x; 1.0378x over previous
import functools

import jax
import jax.numpy as jnp
from jax import lax
from jax.experimental import pallas as pl
from jax.experimental.pallas import tpu as pltpu
from jax.experimental.pallas import tpu_sc as plsc

F32 = jnp.float32
NSUB = 16
NCORE = 2
BATCH = 128
DEGW = 16


def _sc_degree(dstp, zdeg, nacc, eps):
  nb = eps // BATCH
  nw = NCORE * NSUB
  mesh = plsc.VectorSubcoreMesh(core_axis_name="c", subcore_axis_name="s")

  @functools.partial(
      pl.kernel,
      out_type=jax.ShapeDtypeStruct((nw, nacc), F32),
      mesh=mesh,
      scratch_types=[
          pltpu.VMEM((BATCH,), jnp.int32),
          pltpu.VMEM((nacc,), F32),
      ],
      compiler_params=pltpu.CompilerParams(needs_layout_passes=False),
  )
  def deg_kernel(dst_hbm, z_hbm, out_hbm, idxb, hist):
    c = lax.axis_index("c")
    s = lax.axis_index("s")
    w = c * NSUB + s
    pltpu.sync_copy(z_hbm, hist)
    base0 = w * eps

    @pl.loop(0, nb)
    def _(i):
      pltpu.sync_copy(dst_hbm.at[pl.ds(base0 + i * BATCH, BATCH)], idxb)
      for j in range(BATCH // 16):
        d16 = idxb[pl.ds(j * 16, 16)]
        cnt, last = plsc.scan_count(d16)
        old = plsc.load_gather(hist, [d16], mask=last)
        plsc.store_scatter(hist, [d16], old + cnt.astype(F32), mask=last)

    pltpu.sync_copy(hist, out_hbm.at[w])

  return deg_kernel(dstp, zdeg)


def _sc_prop(y, srcp, dstp, zhbm, nacc, eps, d, untiled=False):
  nb = eps // BATCH
  rows_sub = nacc // NSUB
  mesh = plsc.VectorSubcoreMesh(core_axis_name="c", subcore_axis_name="s")
  params = (pltpu.CompilerParams(use_tc_tiling_on_sc=False)
            if untiled else None)

  @functools.partial(
      pl.kernel,
      out_type=jax.ShapeDtypeStruct((NCORE, nacc, d), F32),
      mesh=mesh,
      compiler_params=params,
      scratch_types=(
          [pltpu.VMEM((BATCH,), jnp.int32)] * 2
          + [pltpu.VMEM((BATCH,), jnp.int32)] * 2
          + [pltpu.VMEM((BATCH, d), F32)] * 2
          + [pltpu.VMEM_SHARED((nacc, d), F32)]
          + [pltpu.SemaphoreType.DMA] * 2
      ),
  )
  def prop_kernel(y_hbm, src_hbm, dst_hbm, z_hbm, out_hbm, *scratch):
    srcb = scratch[0:2]
    dstb = scratch[2:4]
    rows = scratch[4:6]
    acc = scratch[6]
    semg = scratch[7:9]
    c = lax.axis_index("c")
    s = lax.axis_index("s")
    w = c * NSUB + s
    pltpu.sync_copy(z_hbm.at[pl.ds(s * rows_sub, rows_sub)],
                    acc.at[pl.ds(s * rows_sub, rows_sub)])
    plsc.subcore_barrier()
    base0 = w * eps

    def load_and_gather(i, q):
      b0 = base0 + i * BATCH
      pltpu.sync_copy(src_hbm.at[pl.ds(b0, BATCH)], srcb[q])
      pltpu.sync_copy(dst_hbm.at[pl.ds(b0, BATCH)], dstb[q])
      pltpu.async_copy(y_hbm.at[srcb[q]], rows[q], semg[q])

    def wait_gather(q):
      pltpu.make_async_copy(y_hbm.at[srcb[q]], rows[q], semg[q]).wait()

    load_and_gather(0, 0)
    load_and_gather(1, 1)

    @pl.loop(0, nb // 2)
    def _(k):
      i = 2 * k
      for q in range(2):
        wait_gather(q)
        pltpu.sync_copy(rows[q], acc.at[dstb[q]], add=True)

        @pl.when(i + 2 + q < nb)
        def _():
          load_and_gather(i + 2 + q, q)

    plsc.subcore_barrier()
    pltpu.sync_copy(acc.at[pl.ds(s * rows_sub, rows_sub)],
                    out_hbm.at[c, pl.ds(s * rows_sub, rows_sub)])

  return prop_kernel(y, srcp, dstp, zhbm)


def _tc_matmul(x, w1, n, nfeat, nhid, rb):
  grid = n // rb

  def body(x_ref, w_ref, h_ref):
    h_ref[...] = jnp.dot(x_ref[...], w_ref[...], preferred_element_type=F32)

  return pl.pallas_call(
      body,
      grid=(grid,),
      in_specs=[
          pl.BlockSpec((rb, nfeat), lambda i: (i, 0)),
          pl.BlockSpec((nfeat, nhid), lambda i: (0, 0)),
      ],
      out_specs=pl.BlockSpec((rb, nhid), lambda i: (i, 0)),
      out_shape=jax.ShapeDtypeStruct((n, nhid), F32),
  )(x, w1)


def _tc_scale1(h1, degp, n, nhid, rb):
  grid = n // rb
  nw = degp.shape[1]

  def body(h_ref, dp_ref, y_ref, dinv_ref):
    ones = jnp.ones((nw, 1), F32)
    deg = jnp.dot(dp_ref[...], ones, preferred_element_type=F32) + 1.0
    dinv = lax.rsqrt(jnp.maximum(deg, 1.0))
    y_ref[...] = dinv * h_ref[...]
    dinv_ref[...] = dinv

  return pl.pallas_call(
      body,
      grid=(grid,),
      in_specs=[
          pl.BlockSpec((rb, nhid), lambda i: (i, 0)),
          pl.BlockSpec((rb, nw), lambda i: (i, 0)),
      ],
      out_specs=[
          pl.BlockSpec((rb, nhid), lambda i: (i, 0)),
          pl.BlockSpec((rb, 1), lambda i: (i, 0)),
      ],
      out_shape=[
          jax.ShapeDtypeStruct((n, nhid), F32),
          jax.ShapeDtypeStruct((n, 1), F32),
      ],
  )(h1, degp)


def _tc_layer2(p0, p1, y1, dinv, b1r, w2p, n, nhid, ncp, rb):
  grid = n // rb

  def body(p0_ref, p1_ref, y1_ref, dinv_ref, b1_ref, w_ref, y2_ref):
    a = p0_ref[...] + p1_ref[...] + y1_ref[...]
    t = jnp.maximum(dinv_ref[...] * a + b1_ref[...], 0.0)
    h2 = jnp.dot(t, w_ref[...], preferred_element_type=F32)
    y2_ref[...] = dinv_ref[...] * h2

  return pl.pallas_call(
      body,
      grid=(grid,),
      in_specs=[
          pl.BlockSpec((rb, nhid), lambda i: (i, 0)),
          pl.BlockSpec((rb, nhid), lambda i: (i, 0)),
          pl.BlockSpec((rb, nhid), lambda i: (i, 0)),
          pl.BlockSpec((rb, 1), lambda i: (i, 0)),
          pl.BlockSpec((1, nhid), lambda i: (0, 0)),
          pl.BlockSpec((nhid, ncp), lambda i: (0, 0)),
      ],
      out_specs=pl.BlockSpec((rb, ncp), lambda i: (i, 0)),
      out_shape=jax.ShapeDtypeStruct((n, ncp), F32),
  )(p0, p1, y1, dinv, b1r, w2p)


def _tc_out(p0, p1, y2, dinv, b2r, n, ncp, ncls, rb):
  grid = n // rb

  def body(p0_ref, p1_ref, y2_ref, dinv_ref, b2_ref, out_ref):
    v = dinv_ref[...] * (p0_ref[...] + p1_ref[...] + y2_ref[...]) + b2_ref[...]
    out_ref[...] = v[:, :ncls]

  return pl.pallas_call(
      body,
      grid=(grid,),
      in_specs=[
          pl.BlockSpec((rb, ncp), lambda i: (i, 0)),
          pl.BlockSpec((rb, ncp), lambda i: (i, 0)),
          pl.BlockSpec((rb, ncp), lambda i: (i, 0)),
          pl.BlockSpec((rb, 1), lambda i: (i, 0)),
          pl.BlockSpec((1, ncp), lambda i: (0, 0)),
      ],
      out_specs=pl.BlockSpec((rb, ncls), lambda i: (i, 0)),
      out_shape=jax.ShapeDtypeStruct((n, ncls), F32),
  )(p0, p1, y2, dinv, b2r)


def kernel(x, edge_index, W1, b1, W2, b2):
  n, nfeat = x.shape
  nhid = W1.shape[1]
  ncls = W2.shape[1]
  e = edge_index.shape[1]
  nw = NCORE * NSUB

  eps = -(-e // (nw * 2 * BATCH)) * 2 * BATCH
  etot = eps * nw
  nacc = -(-(n + BATCH) // 128) * 128
  ncp = 128
  rb = 1000

  ei = edge_index.astype(jnp.int32)
  pad = etot - e
  spread = jnp.arange(pad, dtype=jnp.int32) % BATCH
  srcp = jnp.concatenate([ei[0], spread])
  dstp = jnp.concatenate([ei[1], n + spread])

  zdeg = jnp.zeros((nacc,), F32)
  z1 = jnp.zeros((nacc, nhid), F32)
  z2 = jnp.zeros((nacc, ncp), F32)
  w2p = jnp.pad(W2, ((0, 0), (0, ncp - ncls)))
  b1r = b1.reshape(1, nhid)
  b2r = jnp.pad(b2, (0, ncp - ncls)).reshape(1, ncp)

  degp = _sc_degree(dstp, zdeg, nacc, eps).T
  h1 = _tc_matmul(x, W1, n, nfeat, nhid, rb)
  y1, dinv = _tc_scale1(h1, degp, n, nhid, rb)
  p1 = _sc_prop(y1, srcp, dstp, z1, nacc, eps, nhid)
  y2 = _tc_layer2(p1[0], p1[1], y1, dinv, b1r, w2p, n, nhid, ncp, rb)
  p2 = _sc_prop(y2, srcp, dstp, z2, nacc, eps, ncp)
  return _tc_out(p2[0], p2[1], y2, dinv, b2r, n, ncp, ncls, rb)


# speedup vs baseline: 1.2280x; 1.0602x over previous
import functools

import jax
import jax.numpy as jnp
from jax import lax
from jax.experimental import pallas as pl
from jax.experimental.pallas import tpu as pltpu
from jax.experimental.pallas import tpu_sc as plsc

F32 = jnp.float32
NSUB = 16
NCORE = 2
BATCH = 128
DEGW = 16


def _sc_degree(dstp, zdeg, nacc, eps):
  nb = eps // BATCH
  nw = NCORE * NSUB
  mesh = plsc.VectorSubcoreMesh(core_axis_name="c", subcore_axis_name="s")

  @functools.partial(
      pl.kernel,
      out_type=jax.ShapeDtypeStruct((nw, nacc), F32),
      mesh=mesh,
      scratch_types=[
          pltpu.VMEM((BATCH,), jnp.int32),
          pltpu.VMEM((BATCH,), jnp.int32),
          pltpu.VMEM((nacc,), F32),
          pltpu.SemaphoreType.DMA,
          pltpu.SemaphoreType.DMA,
      ],
      compiler_params=pltpu.CompilerParams(needs_layout_passes=False),
  )
  def deg_kernel(dst_hbm, z_hbm, out_hbm, idxb0, idxb1, hist, sem0, sem1):
    idxb = (idxb0, idxb1)
    sem = (sem0, sem1)
    c = lax.axis_index("c")
    s = lax.axis_index("s")
    w = c * NSUB + s
    pltpu.sync_copy(z_hbm, hist)
    base0 = w * eps

    def start_load(i, q):
      pltpu.async_copy(dst_hbm.at[pl.ds(base0 + i * BATCH, BATCH)],
                       idxb[q], sem[q])

    def histo(q):
      for j in range(BATCH // 16):
        d16 = idxb[q][pl.ds(j * 16, 16)]
        cnt, last = plsc.scan_count(d16)
        old = plsc.load_gather(hist, [d16], mask=last)
        plsc.store_scatter(hist, [d16], old + cnt.astype(F32), mask=last)

    start_load(0, 0)
    start_load(1, 1)

    @pl.loop(0, nb // 2)
    def _(k):
      i = 2 * k
      for q in range(2):
        pltpu.make_async_copy(
            dst_hbm.at[pl.ds(base0 + (i + q) * BATCH, BATCH)],
            idxb[q], sem[q]).wait()
        histo(q)

        @pl.when(i + 2 + q < nb)
        def _():
          start_load(i + 2 + q, q)

    pltpu.sync_copy(hist, out_hbm.at[w])

  return deg_kernel(dstp, zdeg)


def _sc_prop(y, srcp, dstp, zhbm, nacc, eps, d, untiled=False):
  nb = eps // BATCH
  rows_sub = nacc // NSUB
  mesh = plsc.VectorSubcoreMesh(core_axis_name="c", subcore_axis_name="s")
  params = (pltpu.CompilerParams(use_tc_tiling_on_sc=False)
            if untiled else None)

  @functools.partial(
      pl.kernel,
      out_type=jax.ShapeDtypeStruct((NCORE, nacc, d), F32),
      mesh=mesh,
      compiler_params=params,
      scratch_types=(
          [pltpu.VMEM((BATCH,), jnp.int32)] * 2
          + [pltpu.VMEM((BATCH,), jnp.int32)] * 2
          + [pltpu.VMEM((BATCH, d), F32)] * 2
          + [pltpu.VMEM_SHARED((nacc, d), F32)]
          + [pltpu.SemaphoreType.DMA] * 2
      ),
  )
  def prop_kernel(y_hbm, src_hbm, dst_hbm, z_hbm, out_hbm, *scratch):
    srcb = scratch[0:2]
    dstb = scratch[2:4]
    rows = scratch[4:6]
    acc = scratch[6]
    semg = scratch[7:9]
    c = lax.axis_index("c")
    s = lax.axis_index("s")
    w = c * NSUB + s
    pltpu.sync_copy(z_hbm.at[pl.ds(s * rows_sub, rows_sub)],
                    acc.at[pl.ds(s * rows_sub, rows_sub)])
    plsc.subcore_barrier()
    base0 = w * eps

    def load_and_gather(i, q):
      b0 = base0 + i * BATCH
      pltpu.sync_copy(src_hbm.at[pl.ds(b0, BATCH)], srcb[q])
      pltpu.sync_copy(dst_hbm.at[pl.ds(b0, BATCH)], dstb[q])
      pltpu.async_copy(y_hbm.at[srcb[q]], rows[q], semg[q])

    def wait_gather(q):
      pltpu.make_async_copy(y_hbm.at[srcb[q]], rows[q], semg[q]).wait()

    load_and_gather(0, 0)
    load_and_gather(1, 1)

    @pl.loop(0, nb // 2)
    def _(k):
      i = 2 * k
      for q in range(2):
        wait_gather(q)
        pltpu.sync_copy(rows[q], acc.at[dstb[q]], add=True)

        @pl.when(i + 2 + q < nb)
        def _():
          load_and_gather(i + 2 + q, q)

    plsc.subcore_barrier()
    pltpu.sync_copy(acc.at[pl.ds(s * rows_sub, rows_sub)],
                    out_hbm.at[c, pl.ds(s * rows_sub, rows_sub)])

  return prop_kernel(y, srcp, dstp, zhbm)


def _tc_layer1(x, w1, degp, n, nfeat, nhid, rb):
  grid = n // rb
  nw = degp.shape[1]

  def body(x_ref, w_ref, dp_ref, y_ref, dinv_ref):
    ones = jnp.ones((nw, 1), F32)
    deg = jnp.dot(dp_ref[...], ones, preferred_element_type=F32) + 1.0
    dinv = lax.rsqrt(jnp.maximum(deg, 1.0))
    h = jnp.dot(x_ref[...], w_ref[...], preferred_element_type=F32)
    y_ref[...] = dinv * h
    dinv_ref[...] = dinv

  return pl.pallas_call(
      body,
      grid=(grid,),
      in_specs=[
          pl.BlockSpec((rb, nfeat), lambda i: (i, 0)),
          pl.BlockSpec((nfeat, nhid), lambda i: (0, 0)),
          pl.BlockSpec((rb, nw), lambda i: (i, 0)),
      ],
      out_specs=[
          pl.BlockSpec((rb, nhid), lambda i: (i, 0)),
          pl.BlockSpec((rb, 1), lambda i: (i, 0)),
      ],
      out_shape=[
          jax.ShapeDtypeStruct((n, nhid), F32),
          jax.ShapeDtypeStruct((n, 1), F32),
      ],
  )(x, w1, degp)


def _tc_layer2(p0, p1, y1, dinv, b1r, w2p, n, nhid, ncp, rb):
  grid = n // rb

  def body(p0_ref, p1_ref, y1_ref, dinv_ref, b1_ref, w_ref, y2_ref):
    a = p0_ref[...] + p1_ref[...] + y1_ref[...]
    t = jnp.maximum(dinv_ref[...] * a + b1_ref[...], 0.0)
    h2 = jnp.dot(t, w_ref[...], preferred_element_type=F32)
    y2_ref[...] = dinv_ref[...] * h2

  return pl.pallas_call(
      body,
      grid=(grid,),
      in_specs=[
          pl.BlockSpec((rb, nhid), lambda i: (i, 0)),
          pl.BlockSpec((rb, nhid), lambda i: (i, 0)),
          pl.BlockSpec((rb, nhid), lambda i: (i, 0)),
          pl.BlockSpec((rb, 1), lambda i: (i, 0)),
          pl.BlockSpec((1, nhid), lambda i: (0, 0)),
          pl.BlockSpec((nhid, ncp), lambda i: (0, 0)),
      ],
      out_specs=pl.BlockSpec((rb, ncp), lambda i: (i, 0)),
      out_shape=jax.ShapeDtypeStruct((n, ncp), F32),
  )(p0, p1, y1, dinv, b1r, w2p)


def _tc_out(p0, p1, y2, dinv, b2r, n, ncp, ncls, rb):
  grid = n // rb

  def body(p0_ref, p1_ref, y2_ref, dinv_ref, b2_ref, out_ref):
    v = dinv_ref[...] * (p0_ref[...] + p1_ref[...] + y2_ref[...]) + b2_ref[...]
    out_ref[...] = v[:, :ncls]

  return pl.pallas_call(
      body,
      grid=(grid,),
      in_specs=[
          pl.BlockSpec((rb, ncp), lambda i: (i, 0)),
          pl.BlockSpec((rb, ncp), lambda i: (i, 0)),
          pl.BlockSpec((rb, ncp), lambda i: (i, 0)),
          pl.BlockSpec((rb, 1), lambda i: (i, 0)),
          pl.BlockSpec((1, ncp), lambda i: (0, 0)),
      ],
      out_specs=pl.BlockSpec((rb, ncls), lambda i: (i, 0)),
      out_shape=jax.ShapeDtypeStruct((n, ncls), F32),
  )(p0, p1, y2, dinv, b2r)


def kernel(x, edge_index, W1, b1, W2, b2):
  n, nfeat = x.shape
  nhid = W1.shape[1]
  ncls = W2.shape[1]
  e = edge_index.shape[1]
  nw = NCORE * NSUB

  eps = -(-e // (nw * 2 * BATCH)) * 2 * BATCH
  etot = eps * nw
  nacc = -(-(n + BATCH) // 128) * 128
  ncp = 128
  rb = 1000

  ei = edge_index.astype(jnp.int32)
  pad = etot - e
  spread = jnp.arange(pad, dtype=jnp.int32) % BATCH
  srcp = jnp.concatenate([ei[0], spread])
  dstp = jnp.concatenate([ei[1], n + spread])

  zdeg = jnp.zeros((nacc,), F32)
  z1 = jnp.zeros((nacc, nhid), F32)
  z2 = jnp.zeros((nacc, ncp), F32)
  w2p = jnp.pad(W2, ((0, 0), (0, ncp - ncls)))
  b1r = b1.reshape(1, nhid)
  b2r = jnp.pad(b2, (0, ncp - ncls)).reshape(1, ncp)

  degp = _sc_degree(dstp, zdeg, nacc, eps).T
  y1, dinv = _tc_layer1(x, W1, degp, n, nfeat, nhid, rb)
  p1 = _sc_prop(y1, srcp, dstp, z1, nacc, eps, nhid)
  y2 = _tc_layer2(p1[0], p1[1], y1, dinv, b1r, w2p, n, nhid, ncp, rb)
  p2 = _sc_prop(y2, srcp, dstp, z2, nacc, eps, ncp)
  return _tc_out(p2[0], p2[1], y2, dinv, b2r, n, ncp, ncls, rb)


# speedup vs baseline: 1.2285x; 1.0004x over previous
import functools

import jax
import jax.numpy as jnp
from jax import lax
from jax.experimental import pallas as pl
from jax.experimental.pallas import tpu as pltpu
from jax.experimental.pallas import tpu_sc as plsc

F32 = jnp.float32
NSUB = 16
NCORE = 2
BATCH = 128


def _sc_degree(dstp, zdeg, nacc, eps):
  nb = eps // BATCH
  nw = NCORE * NSUB
  mesh = plsc.VectorSubcoreMesh(core_axis_name="c", subcore_axis_name="s")

  @functools.partial(
      pl.kernel,
      out_type=jax.ShapeDtypeStruct((nw, nacc), F32),
      mesh=mesh,
      scratch_types=[
          pltpu.VMEM((BATCH,), jnp.int32),
          pltpu.VMEM((BATCH,), jnp.int32),
          pltpu.VMEM((nacc,), F32),
          pltpu.SemaphoreType.DMA,
          pltpu.SemaphoreType.DMA,
      ],
      compiler_params=pltpu.CompilerParams(needs_layout_passes=False),
  )
  def deg_kernel(dst_hbm, z_hbm, out_hbm, idxb0, idxb1, hist, sem0, sem1):
    idxb = (idxb0, idxb1)
    sem = (sem0, sem1)
    c = lax.axis_index("c")
    s = lax.axis_index("s")
    w = c * NSUB + s
    pltpu.sync_copy(z_hbm, hist)
    base0 = w * eps

    def start_load(i, q):
      pltpu.async_copy(dst_hbm.at[pl.ds(base0 + i * BATCH, BATCH)],
                       idxb[q], sem[q])

    def histo(q):
      for j in range(BATCH // 16):
        d16 = idxb[q][pl.ds(j * 16, 16)]
        cnt, last = plsc.scan_count(d16)
        old = plsc.load_gather(hist, [d16], mask=last)
        plsc.store_scatter(hist, [d16], old + cnt.astype(F32), mask=last)

    start_load(0, 0)
    start_load(1, 1)

    @pl.loop(0, nb // 2)
    def _(k):
      i = 2 * k
      for q in range(2):
        pltpu.make_async_copy(
            dst_hbm.at[pl.ds(base0 + (i + q) * BATCH, BATCH)],
            idxb[q], sem[q]).wait()
        histo(q)

        @pl.when(i + 2 + q < nb)
        def _():
          start_load(i + 2 + q, q)

    pltpu.sync_copy(hist, out_hbm.at[w])

  return deg_kernel(dstp, zdeg)


def _sc_prop(y, srcp, dstp, zhbm, nacc, eps, d):
  nb = eps // BATCH
  rows_sub = nacc // NSUB
  mesh = plsc.VectorSubcoreMesh(core_axis_name="c", subcore_axis_name="s")

  @functools.partial(
      pl.kernel,
      out_type=jax.ShapeDtypeStruct((NCORE, nacc, d), F32),
      mesh=mesh,
      scratch_types=(
          [pltpu.VMEM((BATCH,), jnp.int32)] * 2
          + [pltpu.VMEM((BATCH,), jnp.int32)] * 2
          + [pltpu.VMEM((BATCH, d), F32)] * 2
          + [pltpu.VMEM_SHARED((nacc, d), F32)]
          + [pltpu.SemaphoreType.DMA] * 2
      ),
  )
  def prop_kernel(y_hbm, src_hbm, dst_hbm, z_hbm, out_hbm, *scratch):
    srcb = scratch[0:2]
    dstb = scratch[2:4]
    rows = scratch[4:6]
    acc = scratch[6]
    semg = scratch[7:9]
    c = lax.axis_index("c")
    s = lax.axis_index("s")
    w = c * NSUB + s
    pltpu.sync_copy(z_hbm.at[pl.ds(s * rows_sub, rows_sub)],
                    acc.at[pl.ds(s * rows_sub, rows_sub)])
    plsc.subcore_barrier()
    base0 = w * eps

    def load_and_gather(i, q):
      b0 = base0 + i * BATCH
      pltpu.sync_copy(src_hbm.at[pl.ds(b0, BATCH)], srcb[q])
      pltpu.sync_copy(dst_hbm.at[pl.ds(b0, BATCH)], dstb[q])
      pltpu.async_copy(y_hbm.at[srcb[q]], rows[q], semg[q])

    def wait_gather(q):
      pltpu.make_async_copy(y_hbm.at[srcb[q]], rows[q], semg[q]).wait()

    load_and_gather(0, 0)
    load_and_gather(1, 1)

    @pl.loop(0, nb // 2)
    def _(k):
      i = 2 * k
      for q in range(2):
        wait_gather(q)
        pltpu.sync_copy(rows[q], acc.at[dstb[q]], add=True)

        @pl.when(i + 2 + q < nb)
        def _():
          load_and_gather(i + 2 + q, q)

    plsc.subcore_barrier()
    pltpu.sync_copy(acc.at[pl.ds(s * rows_sub, rows_sub)],
                    out_hbm.at[c, pl.ds(s * rows_sub, rows_sub)])

  return prop_kernel(y, srcp, dstp, zhbm)


def _tc_layer1(x, w1, degp, n, nfeat, nhid, rb):
  grid = n // rb
  nw = degp.shape[1]

  def body(x_ref, w_ref, dp_ref, y_ref, dinv_ref):
    ones = jnp.ones((nw, 1), F32)
    deg = jnp.dot(dp_ref[...], ones, preferred_element_type=F32) + 1.0
    dinv = lax.rsqrt(jnp.maximum(deg, 1.0))
    h = jnp.dot(x_ref[...], w_ref[...], preferred_element_type=F32)
    y_ref[...] = dinv * h
    dinv_ref[...] = dinv

  return pl.pallas_call(
      body,
      grid=(grid,),
      in_specs=[
          pl.BlockSpec((rb, nfeat), lambda i: (i, 0)),
          pl.BlockSpec((nfeat, nhid), lambda i: (0, 0)),
          pl.BlockSpec((rb, nw), lambda i: (i, 0)),
      ],
      out_specs=[
          pl.BlockSpec((rb, nhid), lambda i: (i, 0)),
          pl.BlockSpec((rb, 1), lambda i: (i, 0)),
      ],
      out_shape=[
          jax.ShapeDtypeStruct((n, nhid), F32),
          jax.ShapeDtypeStruct((n, 1), F32),
      ],
  )(x, w1, degp)


def _tc_layer2(p0, p1, y1, dinv, b1r, w2p, n, nhid, ncp, rb):
  grid = n // rb

  def body(p0_ref, p1_ref, y1_ref, dinv_ref, b1_ref, w_ref, y2_ref):
    a = p0_ref[...] + p1_ref[...] + y1_ref[...]
    t = jnp.maximum(dinv_ref[...] * a + b1_ref[...], 0.0)
    h2 = jnp.dot(t, w_ref[...], preferred_element_type=F32)
    y2_ref[...] = dinv_ref[...] * h2

  return pl.pallas_call(
      body,
      grid=(grid,),
      in_specs=[
          pl.BlockSpec((rb, nhid), lambda i: (i, 0)),
          pl.BlockSpec((rb, nhid), lambda i: (i, 0)),
          pl.BlockSpec((rb, nhid), lambda i: (i, 0)),
          pl.BlockSpec((rb, 1), lambda i: (i, 0)),
          pl.BlockSpec((1, nhid), lambda i: (0, 0)),
          pl.BlockSpec((nhid, ncp), lambda i: (0, 0)),
      ],
      out_specs=pl.BlockSpec((rb, ncp), lambda i: (i, 0)),
      out_shape=jax.ShapeDtypeStruct((n, ncp), F32),
  )(p0, p1, y1, dinv, b1r, w2p)


def _tc_out(p0, p1, y2, dinv, b2r, n, ncp, ncls, rb):
  grid = n // rb

  def body(p0_ref, p1_ref, y2_ref, dinv_ref, b2_ref, out_ref):
    v = dinv_ref[...] * (p0_ref[...] + p1_ref[...] + y2_ref[...]) + b2_ref[...]
    out_ref[...] = v[:, :ncls]

  return pl.pallas_call(
      body,
      grid=(grid,),
      in_specs=[
          pl.BlockSpec((rb, ncp), lambda i: (i, 0)),
          pl.BlockSpec((rb, ncp), lambda i: (i, 0)),
          pl.BlockSpec((rb, ncp), lambda i: (i, 0)),
          pl.BlockSpec((rb, 1), lambda i: (i, 0)),
          pl.BlockSpec((1, ncp), lambda i: (0, 0)),
      ],
      out_specs=pl.BlockSpec((rb, ncls), lambda i: (i, 0)),
      out_shape=jax.ShapeDtypeStruct((n, ncls), F32),
  )(p0, p1, y2, dinv, b2r)


def kernel(x, edge_index, W1, b1, W2, b2):
  n, nfeat = x.shape
  nhid = W1.shape[1]
  ncls = W2.shape[1]
  e = edge_index.shape[1]
  nw = NCORE * NSUB

  eps = -(-e // (nw * 2 * BATCH)) * 2 * BATCH
  etot = eps * nw
  nacc = -(-(n + BATCH) // 128) * 128
  ncp = 128
  rb = 1000

  ei = edge_index.astype(jnp.int32)
  pad = etot - e
  spread = jnp.arange(pad, dtype=jnp.int32) % BATCH
  srcp = jnp.concatenate([ei[0], spread])
  dstp = jnp.concatenate([ei[1], n + spread])

  zdeg = jnp.zeros((nacc,), F32)
  z1 = jnp.zeros((nacc, nhid), F32)
  z2 = jnp.zeros((nacc, ncp), F32)
  w2p = jnp.pad(W2, ((0, 0), (0, ncp - ncls)))
  b1r = b1.reshape(1, nhid)
  b2r = jnp.pad(b2, (0, ncp - ncls)).reshape(1, ncp)

  degp = _sc_degree(dstp, zdeg, nacc, eps).T
  y1, dinv = _tc_layer1(x, W1, degp, n, nfeat, nhid, rb)
  p1 = _sc_prop(y1, srcp, dstp, z1, nacc, eps, nhid)
  y2 = _tc_layer2(p1[0], p1[1], y1, dinv, b1r, w2p, n, nhid, ncp, rb)
  p2 = _sc_prop(y2, srcp, dstp, z2, nacc, eps, ncp)
  return _tc_out(p2[0], p2[1], y2, dinv, b2r, n, ncp, ncls, rb)


# speedup vs baseline: 1.4024x; 1.1415x over previous
import functools

import jax
import jax.numpy as jnp
from jax import lax
from jax.experimental import pallas as pl
from jax.experimental.pallas import tpu as pltpu
from jax.experimental.pallas import tpu_sc as plsc

F32 = jnp.float32
NSUB = 16
NCORE = 2
BATCH = 128


def _sc_degree(dstp, zdeg, nacc, eps):
  nb = eps // BATCH
  nw = NCORE * NSUB
  mesh = plsc.VectorSubcoreMesh(core_axis_name="c", subcore_axis_name="s")

  @functools.partial(
      pl.kernel,
      out_type=jax.ShapeDtypeStruct((nw, nacc), F32),
      mesh=mesh,
      scratch_types=[
          pltpu.VMEM((BATCH,), jnp.int32),
          pltpu.VMEM((BATCH,), jnp.int32),
          pltpu.VMEM((nacc,), F32),
          pltpu.SemaphoreType.DMA,
          pltpu.SemaphoreType.DMA,
      ],
      compiler_params=pltpu.CompilerParams(needs_layout_passes=False),
  )
  def deg_kernel(dst_hbm, z_hbm, out_hbm, idxb0, idxb1, hist, sem0, sem1):
    idxb = (idxb0, idxb1)
    sem = (sem0, sem1)
    c = lax.axis_index("c")
    s = lax.axis_index("s")
    w = c * NSUB + s
    pltpu.sync_copy(z_hbm, hist)
    base0 = w * eps

    def start_load(i, q):
      pltpu.async_copy(dst_hbm.at[pl.ds(base0 + i * BATCH, BATCH)],
                       idxb[q], sem[q])

    def histo(q):
      for j in range(BATCH // 16):
        d16 = idxb[q][pl.ds(j * 16, 16)]
        cnt, last = plsc.scan_count(d16)
        old = plsc.load_gather(hist, [d16], mask=last)
        plsc.store_scatter(hist, [d16], old + cnt.astype(F32), mask=last)

    start_load(0, 0)
    start_load(1, 1)

    @pl.loop(0, nb // 2)
    def _(k):
      i = 2 * k
      for q in range(2):
        pltpu.make_async_copy(
            dst_hbm.at[pl.ds(base0 + (i + q) * BATCH, BATCH)],
            idxb[q], sem[q]).wait()
        histo(q)

        @pl.when(i + 2 + q < nb)
        def _():
          start_load(i + 2 + q, q)

    pltpu.sync_copy(hist, out_hbm.at[w])

  return deg_kernel(dstp, zdeg)


def _sc_prop(y, sd, zhbm, nacc, eps, d):
  nb = eps // BATCH
  rows_sub = nacc // NSUB
  mesh = plsc.VectorSubcoreMesh(core_axis_name="c", subcore_axis_name="s")

  @functools.partial(
      pl.kernel,
      out_type=jax.ShapeDtypeStruct((NCORE, nacc, d), F32),
      mesh=mesh,
      scratch_types=(
          [pltpu.VMEM((2, BATCH), jnp.int32)] * 2
          + [pltpu.VMEM((BATCH, d), F32)] * 2
          + [pltpu.VMEM_SHARED((nacc, d), F32)]
          + [pltpu.SemaphoreType.DMA] * 2
      ),
  )
  def prop_kernel(y_hbm, sd_hbm, z_hbm, out_hbm, *scratch):
    idx = scratch[0:2]
    rows = scratch[2:4]
    acc = scratch[4]
    semg = scratch[5:7]
    c = lax.axis_index("c")
    s = lax.axis_index("s")
    w = c * NSUB + s
    pltpu.sync_copy(z_hbm.at[pl.ds(s * rows_sub, rows_sub)],
                    acc.at[pl.ds(s * rows_sub, rows_sub)])
    plsc.subcore_barrier()
    bbase = w * nb

    def load_and_gather(i, q):
      pltpu.sync_copy(sd_hbm.at[bbase + i], idx[q])
      pltpu.async_copy(y_hbm.at[idx[q].at[0]], rows[q], semg[q])

    def wait_gather(q):
      pltpu.make_async_copy(y_hbm.at[idx[q].at[0]], rows[q], semg[q]).wait()

    load_and_gather(0, 0)
    load_and_gather(1, 1)

    @pl.loop(0, nb // 2)
    def _(k):
      i = 2 * k
      for q in range(2):
        wait_gather(q)
        pltpu.sync_copy(rows[q], acc.at[idx[q].at[1]], add=True)

        @pl.when(i + 2 + q < nb)
        def _():
          load_and_gather(i + 2 + q, q)

    plsc.subcore_barrier()
    pltpu.sync_copy(acc.at[pl.ds(s * rows_sub, rows_sub)],
                    out_hbm.at[c, pl.ds(s * rows_sub, rows_sub)])

  return prop_kernel(y, sd, zhbm)


def _tc_layer1(x, w1, degp, n, nfeat, nhid, rb):
  grid = n // rb
  nw = degp.shape[1]

  def body(x_ref, w_ref, dp_ref, y_ref, dinv_ref):
    ones = jnp.ones((nw, 1), F32)
    deg = jnp.dot(dp_ref[...], ones, preferred_element_type=F32) + 1.0
    dinv = lax.rsqrt(jnp.maximum(deg, 1.0))
    h = jnp.dot(x_ref[...], w_ref[...], preferred_element_type=F32)
    y_ref[...] = dinv * h
    dinv_ref[...] = dinv

  return pl.pallas_call(
      body,
      grid=(grid,),
      in_specs=[
          pl.BlockSpec((rb, nfeat), lambda i: (i, 0)),
          pl.BlockSpec((nfeat, nhid), lambda i: (0, 0)),
          pl.BlockSpec((rb, nw), lambda i: (i, 0)),
      ],
      out_specs=[
          pl.BlockSpec((rb, nhid), lambda i: (i, 0)),
          pl.BlockSpec((rb, 1), lambda i: (i, 0)),
      ],
      out_shape=[
          jax.ShapeDtypeStruct((n, nhid), F32),
          jax.ShapeDtypeStruct((n, 1), F32),
      ],
  )(x, w1, degp)


def _tc_layer2(p0, p1, y1, dinv, b1r, w2p, n, nhid, ncp, rb):
  grid = n // rb

  def body(p0_ref, p1_ref, y1_ref, dinv_ref, b1_ref, w_ref, y2_ref):
    a = p0_ref[...] + p1_ref[...] + y1_ref[...]
    t = jnp.maximum(dinv_ref[...] * a + b1_ref[...], 0.0)
    h2 = jnp.dot(t, w_ref[...], preferred_element_type=F32)
    y2_ref[...] = dinv_ref[...] * h2

  return pl.pallas_call(
      body,
      grid=(grid,),
      in_specs=[
          pl.BlockSpec((rb, nhid), lambda i: (i, 0)),
          pl.BlockSpec((rb, nhid), lambda i: (i, 0)),
          pl.BlockSpec((rb, nhid), lambda i: (i, 0)),
          pl.BlockSpec((rb, 1), lambda i: (i, 0)),
          pl.BlockSpec((1, nhid), lambda i: (0, 0)),
          pl.BlockSpec((nhid, ncp), lambda i: (0, 0)),
      ],
      out_specs=pl.BlockSpec((rb, ncp), lambda i: (i, 0)),
      out_shape=jax.ShapeDtypeStruct((n, ncp), F32),
  )(p0, p1, y1, dinv, b1r, w2p)


def _tc_out(p0, p1, y2, dinv, b2r, n, ncp, ncls, rb):
  grid = n // rb

  def body(p0_ref, p1_ref, y2_ref, dinv_ref, b2_ref, out_ref):
    v = dinv_ref[...] * (p0_ref[...] + p1_ref[...] + y2_ref[...]) + b2_ref[...]
    out_ref[...] = v[:, :ncls]

  return pl.pallas_call(
      body,
      grid=(grid,),
      in_specs=[
          pl.BlockSpec((rb, ncp), lambda i: (i, 0)),
          pl.BlockSpec((rb, ncp), lambda i: (i, 0)),
          pl.BlockSpec((rb, ncp), lambda i: (i, 0)),
          pl.BlockSpec((rb, 1), lambda i: (i, 0)),
          pl.BlockSpec((1, ncp), lambda i: (0, 0)),
      ],
      out_specs=pl.BlockSpec((rb, ncls), lambda i: (i, 0)),
      out_shape=jax.ShapeDtypeStruct((n, ncls), F32),
  )(p0, p1, y2, dinv, b2r)


def kernel(x, edge_index, W1, b1, W2, b2):
  n, nfeat = x.shape
  nhid = W1.shape[1]
  ncls = W2.shape[1]
  e = edge_index.shape[1]
  nw = NCORE * NSUB

  eps = -(-e // (nw * 2 * BATCH)) * 2 * BATCH
  etot = eps * nw
  nacc = -(-(n + BATCH) // 128) * 128
  ncp = 128
  rb = 1000

  ei = edge_index.astype(jnp.int32)
  pad = etot - e
  spread = jnp.arange(pad, dtype=jnp.int32) % BATCH
  srcp = jnp.concatenate([ei[0], spread])
  dstp = jnp.concatenate([ei[1], n + spread])
  sd = jnp.stack([srcp.reshape(-1, BATCH), dstp.reshape(-1, BATCH)], axis=1)

  zdeg = jnp.zeros((nacc,), F32)
  z1 = jnp.zeros((nacc, nhid), F32)
  z2 = jnp.zeros((nacc, ncp), F32)
  w2p = jnp.pad(W2, ((0, 0), (0, ncp - ncls)))
  b1r = b1.reshape(1, nhid)
  b2r = jnp.pad(b2, (0, ncp - ncls)).reshape(1, ncp)

  degp = _sc_degree(dstp, zdeg, nacc, eps).T
  y1, dinv = _tc_layer1(x, W1, degp, n, nfeat, nhid, rb)
  p1 = _sc_prop(y1, sd, z1, nacc, eps, nhid)
  y2 = _tc_layer2(p1[0], p1[1], y1, dinv, b1r, w2p, n, nhid, ncp, rb)
  p2 = _sc_prop(y2, sd, z2, nacc, eps, ncp)
  return _tc_out(p2[0], p2[1], y2, dinv, b2r, n, ncp, ncls, rb)


# speedup vs baseline: 1.5307x; 1.0915x over previous
import functools

import jax
import jax.numpy as jnp
from jax import lax
from jax.experimental import pallas as pl
from jax.experimental.pallas import tpu as pltpu
from jax.experimental.pallas import tpu_sc as plsc

F32 = jnp.float32
NSUB = 16
NCORE = 2
BATCH = 128


def _sc_degree(dstp, zdeg, nacc, eps):
  nb = eps // BATCH
  nw = NCORE * NSUB
  mesh = plsc.VectorSubcoreMesh(core_axis_name="c", subcore_axis_name="s")

  @functools.partial(
      pl.kernel,
      out_type=jax.ShapeDtypeStruct((nw, nacc), F32),
      mesh=mesh,
      scratch_types=[
          pltpu.VMEM((BATCH,), jnp.int32),
          pltpu.VMEM((BATCH,), jnp.int32),
          pltpu.VMEM((nacc,), F32),
          pltpu.SemaphoreType.DMA,
          pltpu.SemaphoreType.DMA,
      ],
      compiler_params=pltpu.CompilerParams(needs_layout_passes=False),
  )
  def deg_kernel(dst_hbm, z_hbm, out_hbm, idxb0, idxb1, hist, sem0, sem1):
    idxb = (idxb0, idxb1)
    sem = (sem0, sem1)
    c = lax.axis_index("c")
    s = lax.axis_index("s")
    w = c * NSUB + s
    pltpu.sync_copy(z_hbm, hist)
    base0 = w * eps

    def start_load(i, q):
      pltpu.async_copy(dst_hbm.at[pl.ds(base0 + i * BATCH, BATCH)],
                       idxb[q], sem[q])

    def histo(q):
      for j in range(BATCH // 16):
        d16 = idxb[q][pl.ds(j * 16, 16)]
        cnt, last = plsc.scan_count(d16)
        old = plsc.load_gather(hist, [d16], mask=last)
        plsc.store_scatter(hist, [d16], old + cnt.astype(F32), mask=last)

    start_load(0, 0)
    start_load(1, 1)

    @pl.loop(0, nb // 2)
    def _(k):
      i = 2 * k
      for q in range(2):
        pltpu.make_async_copy(
            dst_hbm.at[pl.ds(base0 + (i + q) * BATCH, BATCH)],
            idxb[q], sem[q]).wait()
        histo(q)

        @pl.when(i + 2 + q < nb)
        def _():
          start_load(i + 2 + q, q)

    pltpu.sync_copy(hist, out_hbm.at[w])

  return deg_kernel(dstp, zdeg)


def _sc_prop(y, sd, zhbm, nacc, eps, d):
  nb = eps // BATCH
  rows_sub = nacc // NSUB
  mesh = plsc.VectorSubcoreMesh(core_axis_name="c", subcore_axis_name="s")

  @functools.partial(
      pl.kernel,
      out_type=jax.ShapeDtypeStruct((NCORE, nacc, d), F32),
      mesh=mesh,
      scratch_types=(
          [pltpu.VMEM((2, BATCH), jnp.int32)] * 4
          + [pltpu.VMEM((BATCH, d), F32)] * 2
          + [pltpu.VMEM_SHARED((nacc, d), F32)]
          + [pltpu.SemaphoreType.DMA] * 6
      ),
  )
  def prop_kernel(y_hbm, sd_hbm, z_hbm, out_hbm, *scratch):
    idx = scratch[0:4]
    rows = scratch[4:6]
    acc = scratch[6]
    semg = scratch[7:9]
    semi = scratch[9:13]
    c = lax.axis_index("c")
    s = lax.axis_index("s")
    w = c * NSUB + s
    pltpu.sync_copy(z_hbm.at[pl.ds(s * rows_sub, rows_sub)],
                    acc.at[pl.ds(s * rows_sub, rows_sub)])
    plsc.subcore_barrier()
    bbase = w * nb

    def start_idx(i, r):
      pltpu.async_copy(sd_hbm.at[bbase + i], idx[r], semi[r])

    def wait_idx(i, r):
      pltpu.make_async_copy(sd_hbm.at[bbase + i], idx[r], semi[r]).wait()

    def start_gather(i, r, q):
      wait_idx(i, r)
      pltpu.async_copy(y_hbm.at[idx[r].at[0]], rows[q], semg[q])

    def wait_gather(r, q):
      pltpu.make_async_copy(y_hbm.at[idx[r].at[0]], rows[q], semg[q]).wait()

    for r in range(4):
      start_idx(r, r)
    start_gather(0, 0, 0)
    start_gather(1, 1, 1)

    @pl.loop(0, nb // 4)
    def _(k):
      for j in range(4):
        i = 4 * k + j
        r = j
        q = j % 2
        wait_gather(r, q)
        pltpu.sync_copy(rows[q], acc.at[idx[r].at[1]], add=True)

        @pl.when(i + 4 < nb)
        def _():
          start_idx(i + 4, r)

        @pl.when(i + 2 < nb)
        def _():
          start_gather(i + 2, (j + 2) % 4, q)

    plsc.subcore_barrier()
    pltpu.sync_copy(acc.at[pl.ds(s * rows_sub, rows_sub)],
                    out_hbm.at[c, pl.ds(s * rows_sub, rows_sub)])

  return prop_kernel(y, sd, zhbm)


def _tc_layer1(x, w1, degp, n, nfeat, nhid, rb):
  grid = n // rb
  nw = degp.shape[1]

  def body(x_ref, w_ref, dp_ref, y_ref, dinv_ref):
    ones = jnp.ones((nw, 1), F32)
    deg = jnp.dot(dp_ref[...], ones, preferred_element_type=F32) + 1.0
    dinv = lax.rsqrt(jnp.maximum(deg, 1.0))
    h = jnp.dot(x_ref[...], w_ref[...], preferred_element_type=F32)
    y_ref[...] = dinv * h
    dinv_ref[...] = dinv

  return pl.pallas_call(
      body,
      grid=(grid,),
      in_specs=[
          pl.BlockSpec((rb, nfeat), lambda i: (i, 0)),
          pl.BlockSpec((nfeat, nhid), lambda i: (0, 0)),
          pl.BlockSpec((rb, nw), lambda i: (i, 0)),
      ],
      out_specs=[
          pl.BlockSpec((rb, nhid), lambda i: (i, 0)),
          pl.BlockSpec((rb, 1), lambda i: (i, 0)),
      ],
      out_shape=[
          jax.ShapeDtypeStruct((n, nhid), F32),
          jax.ShapeDtypeStruct((n, 1), F32),
      ],
  )(x, w1, degp)


def _tc_layer2(p0, p1, y1, dinv, b1r, w2p, n, nhid, ncp, rb):
  grid = n // rb

  def body(p0_ref, p1_ref, y1_ref, dinv_ref, b1_ref, w_ref, y2_ref):
    a = p0_ref[...] + p1_ref[...] + y1_ref[...]
    t = jnp.maximum(dinv_ref[...] * a + b1_ref[...], 0.0)
    h2 = jnp.dot(t, w_ref[...], preferred_element_type=F32)
    y2_ref[...] = dinv_ref[...] * h2

  return pl.pallas_call(
      body,
      grid=(grid,),
      in_specs=[
          pl.BlockSpec((rb, nhid), lambda i: (i, 0)),
          pl.BlockSpec((rb, nhid), lambda i: (i, 0)),
          pl.BlockSpec((rb, nhid), lambda i: (i, 0)),
          pl.BlockSpec((rb, 1), lambda i: (i, 0)),
          pl.BlockSpec((1, nhid), lambda i: (0, 0)),
          pl.BlockSpec((nhid, ncp), lambda i: (0, 0)),
      ],
      out_specs=pl.BlockSpec((rb, ncp), lambda i: (i, 0)),
      out_shape=jax.ShapeDtypeStruct((n, ncp), F32),
  )(p0, p1, y1, dinv, b1r, w2p)


def _tc_out(p0, p1, y2, dinv, b2r, n, ncp, ncls, rb):
  grid = n // rb

  def body(p0_ref, p1_ref, y2_ref, dinv_ref, b2_ref, out_ref):
    v = dinv_ref[...] * (p0_ref[...] + p1_ref[...] + y2_ref[...]) + b2_ref[...]
    out_ref[...] = v[:, :ncls]

  return pl.pallas_call(
      body,
      grid=(grid,),
      in_specs=[
          pl.BlockSpec((rb, ncp), lambda i: (i, 0)),
          pl.BlockSpec((rb, ncp), lambda i: (i, 0)),
          pl.BlockSpec((rb, ncp), lambda i: (i, 0)),
          pl.BlockSpec((rb, 1), lambda i: (i, 0)),
          pl.BlockSpec((1, ncp), lambda i: (0, 0)),
      ],
      out_specs=pl.BlockSpec((rb, ncls), lambda i: (i, 0)),
      out_shape=jax.ShapeDtypeStruct((n, ncls), F32),
  )(p0, p1, y2, dinv, b2r)


def kernel(x, edge_index, W1, b1, W2, b2):
  n, nfeat = x.shape
  nhid = W1.shape[1]
  ncls = W2.shape[1]
  e = edge_index.shape[1]
  nw = NCORE * NSUB

  eps = -(-e // (nw * 4 * BATCH)) * 4 * BATCH
  etot = eps * nw
  nacc = -(-(n + BATCH) // 128) * 128
  ncp = 128
  rb = 1000

  ei = edge_index.astype(jnp.int32)
  pad = etot - e
  spread = jnp.arange(pad, dtype=jnp.int32) % BATCH
  srcp = jnp.concatenate([ei[0], spread])
  dstp = jnp.concatenate([ei[1], n + spread])
  sd = jnp.stack([srcp.reshape(-1, BATCH), dstp.reshape(-1, BATCH)], axis=1)

  zdeg = jnp.zeros((nacc,), F32)
  z1 = jnp.zeros((nacc, nhid), F32)
  z2 = jnp.zeros((nacc, ncp), F32)
  w2p = jnp.pad(W2, ((0, 0), (0, ncp - ncls)))
  b1r = b1.reshape(1, nhid)
  b2r = jnp.pad(b2, (0, ncp - ncls)).reshape(1, ncp)

  degp = _sc_degree(dstp, zdeg, nacc, eps).T
  y1, dinv = _tc_layer1(x, W1, degp, n, nfeat, nhid, rb)
  p1 = _sc_prop(y1, sd, z1, nacc, eps, nhid)
  y2 = _tc_layer2(p1[0], p1[1], y1, dinv, b1r, w2p, n, nhid, ncp, rb)
  p2 = _sc_prop(y2, sd, z2, nacc, eps, ncp)
  return _tc_out(p2[0], p2[1], y2, dinv, b2r, n, ncp, ncls, rb)
